# baseline (speedup 1.0000x reference)
import jax
import jax.numpy as jnp
from jax import lax
from jax.experimental import pallas as pl
from jax.experimental.pallas import tpu as pltpu


def _moe_dense_kernel(lg_ref, x_ref, w0_ref, b0_ref, w1_ref, b1_ref, o_ref,
                      xb_ref):
    h = pl.program_id(0)

    @pl.when(h == 0)
    def _cast_x():
        xb_ref[...] = x_ref[...].astype(jnp.bfloat16)

    xb = xb_ref[...]
    w0b = w0_ref[...].astype(jnp.bfloat16)
    w1b = w1_ref[...].astype(jnp.bfloat16)
    h0 = lax.dot_general(xb, w0b, (((1,), (1,)), ((), ())),
                         preferred_element_type=jnp.float32)
    h1 = lax.dot_general(xb, w1b, (((1,), (1,)), ((), ())),
                         preferred_element_type=jnp.float32)
    h0 = h0 + b0_ref[0, 0, :][None, :]
    h1 = h1 + b1_ref[0, 0, :][None, :]
    e1 = lg_ref[:, 1] > lg_ref[:, 0]
    o_ref[...] = jnp.where(e1[:, None], h1, h0)


def kernel(x, Wg, bg, W0, b0, W1, b1):
    Bb, Nn, C = x.shape
    T = Bb * Nn
    H = W0.shape[0]
    inp = x.reshape(T, C)

    logits = inp @ Wg.T + bg

    TH = min(128, H)
    h_tiles = H // TH

    b0r = b0.reshape(h_tiles, 1, TH)
    b1r = b1.reshape(h_tiles, 1, TH)

    out = pl.pallas_call(
        _moe_dense_kernel,
        grid=(h_tiles,),
        in_specs=[
            pl.BlockSpec((T, 2), lambda h: (0, 0)),
            pl.BlockSpec((T, C), lambda h: (0, 0)),
            pl.BlockSpec((TH, C), lambda h: (h, 0)),
            pl.BlockSpec((1, 1, TH), lambda h: (h, 0, 0)),
            pl.BlockSpec((TH, C), lambda h: (h, 0)),
            pl.BlockSpec((1, 1, TH), lambda h: (h, 0, 0)),
        ],
        out_specs=pl.BlockSpec((T, TH), lambda h: (0, h)),
        out_shape=jax.ShapeDtypeStruct((T, H), jnp.float32),
        scratch_shapes=[
            pltpu.VMEM((T, C), jnp.bfloat16),
        ],
        compiler_params=pltpu.CompilerParams(
            dimension_semantics=("arbitrary",),
            vmem_limit_bytes=100 * 1024 * 1024,
        ),
    )(logits, inp, W0, b0r, W1, b1r)
    return out.reshape(Bb, Nn, H)
